# Initial kernel scaffold; baseline (speedup 1.0000x reference)
import jax
import jax.numpy as jnp
from jax.experimental import pallas as pl


def kernel(x, edge_index, edge_weight, W1, b1, W2, b2):
    raise NotImplementedError("write your pallas kernel here")


# speedup vs baseline: 3.0896x; 3.0896x over previous
import functools

import jax
import jax.numpy as jnp
from jax import lax
from jax.experimental import pallas as pl
from jax.experimental.pallas import tpu as pltpu
from jax.experimental.pallas import tpu_sc as plsc

N_NODES = 10000
N_EDGES = 320000
NFEAT = 128
NHID = 128
NCLASS = 40
NCLS_PAD = 64

NW = 32
EDGES_PER_W = N_EDGES // NW
CHUNK = 80
NCHUNK = EDGES_PER_W // CHUNK
DRAIN = 80
NDRAIN = N_NODES // DRAIN


def _make_spmm(D):
  mesh = plsc.VectorSubcoreMesh(
      core_axis_name="c", subcore_axis_name="s", num_cores=2, num_subcores=16)

  @functools.partial(
      pl.kernel,
      out_type=jax.ShapeDtypeStruct((2, N_NODES, D), jnp.float32),
      mesh=mesh,
      scratch_types=[
          pltpu.VMEM((1, CHUNK), jnp.int32),
          pltpu.VMEM((1, CHUNK), jnp.int32),
          pltpu.VMEM((CHUNK, 16), jnp.float32),
          pltpu.VMEM((CHUNK, D), jnp.float32),
          pltpu.VMEM_SHARED((N_NODES, D), jnp.float32),
          pltpu.SemaphoreType.DMA,
      ],
      compiler_params=pltpu.CompilerParams(use_tc_tiling_on_sc=False),
  )
  def spmm(sup_hbm, src_hbm, dst_hbm, ew_hbm, out_hbm,
           src_v, dst_v, ew_v, rows_v, acc, sem):
    c = lax.axis_index("c")
    s = lax.axis_index("s")
    w = s * 2 + c
    n_rr = jnp.where(s < NDRAIN % 16, NDRAIN // 16 + 1, NDRAIN // 16)

    def zero_rows(i, _):
      for g in range(D // 16):
        rows_v[i, pl.ds(16 * g, 16)] = jnp.zeros((16,), jnp.float32)
      return 0
    lax.fori_loop(0, CHUNK, zero_rows, 0)

    def zero_acc(j, _):
      pltpu.sync_copy(rows_v, acc.at[pl.ds((s + j * 16) * DRAIN, DRAIN)])
      return 0
    lax.fori_loop(0, n_rr, zero_acc, 0)
    plsc.subcore_barrier()

    def chunk_body(k, _):
      pltpu.sync_copy(src_hbm.at[w].at[pl.ds(k, 1)], src_v)
      pltpu.sync_copy(dst_hbm.at[w].at[pl.ds(k, 1)], dst_v)
      pltpu.sync_copy(ew_hbm.at[w].at[pl.ds(k * CHUNK, CHUNK)], ew_v)
      pltpu.async_copy(sup_hbm.at[src_v.at[0]], rows_v, sem).wait()

      def edge_body(i, _):
        w16 = ew_v[i]
        for g in range(D // 16):
          sl = pl.ds(16 * g, 16)
          rows_v[i, sl] = rows_v[i, sl] * w16
        return 0
      lax.fori_loop(0, CHUNK, edge_body, 0)

      pltpu.sync_copy(rows_v, acc.at[dst_v.at[0]], add=True)
      return 0
    lax.fori_loop(0, NCHUNK, chunk_body, 0)

    plsc.subcore_barrier()

    def drain(j, _):
      base = (s + j * 16) * DRAIN
      pltpu.sync_copy(acc.at[pl.ds(base, DRAIN)], rows_v)
      pltpu.sync_copy(rows_v, out_hbm.at[c].at[pl.ds(base, DRAIN)])
      return 0
    lax.fori_loop(0, n_rr, drain, 0)

  return spmm


_spmm128 = _make_spmm(NHID)
_spmm64 = _make_spmm(NCLS_PAD)

_RB = 1000


def _mm1_body(x_ref, w_ref, o_ref):
  o_ref[...] = jnp.dot(x_ref[...], w_ref[...],
                       preferred_element_type=jnp.float32)


def _mm1(x, W1):
  return pl.pallas_call(
      _mm1_body,
      grid=(N_NODES // _RB,),
      in_specs=[
          pl.BlockSpec((_RB, NFEAT), lambda i: (i, 0)),
          pl.BlockSpec((NFEAT, NHID), lambda i: (0, 0)),
      ],
      out_specs=pl.BlockSpec((_RB, NHID), lambda i: (i, 0)),
      out_shape=jax.ShapeDtypeStruct((N_NODES, NHID), jnp.float32),
  )(x, W1)


def _combine_body(p0_ref, p1_ref, b_ref, w_ref, o_ref):
  h = jnp.maximum(p0_ref[...] + p1_ref[...] + b_ref[...], 0.0)
  o_ref[...] = jnp.dot(h, w_ref[...], preferred_element_type=jnp.float32)


def _combine(p0, p1, b1, W2p):
  return pl.pallas_call(
      _combine_body,
      grid=(N_NODES // _RB,),
      in_specs=[
          pl.BlockSpec((_RB, NHID), lambda i: (i, 0)),
          pl.BlockSpec((_RB, NHID), lambda i: (i, 0)),
          pl.BlockSpec((1, NHID), lambda i: (0, 0)),
          pl.BlockSpec((NHID, NCLS_PAD), lambda i: (0, 0)),
      ],
      out_specs=pl.BlockSpec((_RB, NCLS_PAD), lambda i: (i, 0)),
      out_shape=jax.ShapeDtypeStruct((N_NODES, NCLS_PAD), jnp.float32),
  )(p0, p1, b1.reshape(1, NHID), W2p)


def _final_body(p0_ref, p1_ref, b_ref, o_ref):
  z = (p0_ref[...] + p1_ref[...])[:, :NCLASS] + b_ref[...]
  z = z - jnp.max(z, axis=1, keepdims=True)
  e = jnp.exp(z)
  o_ref[...] = e / jnp.sum(e, axis=1, keepdims=True)


def _final(p0, p1, b2):
  return pl.pallas_call(
      _final_body,
      grid=(N_NODES // _RB,),
      in_specs=[
          pl.BlockSpec((_RB, NCLS_PAD), lambda i: (i, 0)),
          pl.BlockSpec((_RB, NCLS_PAD), lambda i: (i, 0)),
          pl.BlockSpec((1, NCLASS), lambda i: (0, 0)),
      ],
      out_specs=pl.BlockSpec((_RB, NCLASS), lambda i: (i, 0)),
      out_shape=jax.ShapeDtypeStruct((N_NODES, NCLASS), jnp.float32),
  )(p0, p1, b2.reshape(1, NCLASS))


@jax.jit
def kernel(x, edge_index, edge_weight, W1, b1, W2, b2):
  src = edge_index[0].astype(jnp.int32).reshape(NW, NCHUNK, CHUNK)
  dst = edge_index[1].astype(jnp.int32).reshape(NW, NCHUNK, CHUNK)
  ew = jnp.broadcast_to(edge_weight.reshape(NW, EDGES_PER_W, 1),
                        (NW, EDGES_PER_W, 16))
  W2p = jnp.pad(W2, ((0, 0), (0, NCLS_PAD - NCLASS)))

  sup1 = _mm1(x, W1)
  p1 = _spmm128(sup1, src, dst, ew)
  sup2 = _combine(p1[0], p1[1], b1, W2p)
  p2 = _spmm64(sup2, src, dst, ew)
  return _final(p2[0], p2[1], b2)


# speedup vs baseline: 3.9758x; 1.2868x over previous
import functools

import jax
import jax.numpy as jnp
from jax import lax
from jax.experimental import pallas as pl
from jax.experimental.pallas import tpu as pltpu
from jax.experimental.pallas import tpu_sc as plsc

N_NODES = 10000
N_EDGES = 320000
NFEAT = 128
NHID = 128
NCLASS = 40
NCLS_PAD = 64

NW = 32
CHUNK = 128
NCHUNK = 80
EDGES_PER_W = CHUNK * NCHUNK
E_PAD = NW * EDGES_PER_W
DRAIN = 80
NDRAIN = N_NODES // DRAIN


def _make_spmm(D):
  mesh = plsc.VectorSubcoreMesh(
      core_axis_name="c", subcore_axis_name="s", num_cores=2, num_subcores=16)

  @functools.partial(
      pl.kernel,
      out_type=jax.ShapeDtypeStruct((2, N_NODES, D), jnp.float32),
      mesh=mesh,
      scratch_types=[
          pltpu.VMEM((2, CHUNK), jnp.int32),
          pltpu.VMEM((2, CHUNK), jnp.int32),
          pltpu.VMEM((CHUNK, 16), jnp.float32),
          pltpu.VMEM((CHUNK, 16), jnp.float32),
          pltpu.VMEM((CHUNK, D), jnp.float32),
          pltpu.VMEM((CHUNK, D), jnp.float32),
          pltpu.SemaphoreType.DMA,
          pltpu.SemaphoreType.DMA,
          pltpu.SemaphoreType.DMA,
          pltpu.SemaphoreType.DMA,
          pltpu.SemaphoreType.DMA,
          pltpu.SemaphoreType.DMA,
          pltpu.VMEM_SHARED((N_NODES, D), jnp.float32),
      ],
      compiler_params=pltpu.CompilerParams(use_tc_tiling_on_sc=False),
  )
  def spmm(sup_hbm, sd_hbm, ew_hbm, out_hbm,
           sd0, sd1, ew0, ew1, rows0, rows1,
           ssd0, ssd1, sew0, sew1, sg0, sg1, acc):
    c = lax.axis_index("c")
    s = lax.axis_index("s")
    w = s * 2 + c
    sd = (sd0, sd1)
    ewv = (ew0, ew1)
    rows = (rows0, rows1)
    ssd = (ssd0, ssd1)
    sew = (sew0, sew1)
    sg = (sg0, sg1)
    n_rr = jnp.where(s < NDRAIN % 16, NDRAIN // 16 + 1, NDRAIN // 16)

    def zero_rows(i, _):
      for g in range(D // 16):
        rows0[i, pl.ds(16 * g, 16)] = jnp.zeros((16,), jnp.float32)
      return 0
    lax.fori_loop(0, DRAIN, zero_rows, 0)

    def zero_acc(j, _):
      pltpu.sync_copy(rows0.at[pl.ds(0, DRAIN)],
                      acc.at[pl.ds((s + j * 16) * DRAIN, DRAIN)])
      return 0
    lax.fori_loop(0, n_rr, zero_acc, 0)
    plsc.subcore_barrier()

    def issue_idx(k, b):
      pltpu.async_copy(sd_hbm.at[w].at[k], sd[b], ssd[b])
      pltpu.async_copy(ew_hbm.at[w].at[pl.ds(k * CHUNK, CHUNK)], ewv[b],
                       sew[b])

    def wait_idx(k, b):
      pltpu.make_async_copy(sd_hbm.at[w].at[k], sd[b], ssd[b]).wait()
      pltpu.make_async_copy(ew_hbm.at[w].at[pl.ds(k * CHUNK, CHUNK)], ewv[b],
                            sew[b]).wait()

    def issue_gather(b):
      pltpu.async_copy(sup_hbm.at[sd[b].at[0]], rows[b], sg[b])

    def wait_gather(b):
      pltpu.make_async_copy(sup_hbm.at[sd[b].at[0]], rows[b], sg[b]).wait()

    def scale_scatter(b):
      def edge_body(i, _):
        for u in range(2):
          row = 2 * i + u
          w16 = ewv[b][row]
          for g in range(D // 16):
            sl = pl.ds(16 * g, 16)
            rows[b][row, sl] = rows[b][row, sl] * w16
        return 0
      lax.fori_loop(0, CHUNK // 2, edge_body, 0)
      pltpu.sync_copy(rows[b], acc.at[sd[b].at[1]], add=True)

    issue_idx(0, 0)
    wait_idx(0, 0)
    issue_idx(1, 1)
    issue_gather(0)

    def pipe_body(j, _):
      for b in range(2):
        k = 2 * j + b
        nb = 1 - b
        wait_idx(k + 1, nb)
        issue_gather(nb)
        wait_gather(b)
        scale_scatter(b)
        issue_idx(k + 2, b)
      return 0
    lax.fori_loop(0, NCHUNK // 2 - 1, pipe_body, 0)

    wait_idx(NCHUNK - 1, 1)
    issue_gather(1)
    wait_gather(0)
    scale_scatter(0)
    wait_gather(1)
    scale_scatter(1)

    plsc.subcore_barrier()

    def drain(j, _):
      base = (s + j * 16) * DRAIN
      pltpu.sync_copy(acc.at[pl.ds(base, DRAIN)], rows0.at[pl.ds(0, DRAIN)])
      pltpu.sync_copy(rows0.at[pl.ds(0, DRAIN)],
                      out_hbm.at[c].at[pl.ds(base, DRAIN)])
      return 0
    lax.fori_loop(0, n_rr, drain, 0)

  return spmm


_spmm128 = _make_spmm(NHID)
_spmm64 = _make_spmm(NCLS_PAD)

_RB = 1000


def _mm1_body(x_ref, w_ref, o_ref):
  o_ref[...] = jnp.dot(x_ref[...], w_ref[...],
                       preferred_element_type=jnp.float32)


def _mm1(x, W1):
  return pl.pallas_call(
      _mm1_body,
      grid=(N_NODES // _RB,),
      in_specs=[
          pl.BlockSpec((_RB, NFEAT), lambda i: (i, 0)),
          pl.BlockSpec((NFEAT, NHID), lambda i: (0, 0)),
      ],
      out_specs=pl.BlockSpec((_RB, NHID), lambda i: (i, 0)),
      out_shape=jax.ShapeDtypeStruct((N_NODES, NHID), jnp.float32),
  )(x, W1)


def _combine_body(p0_ref, p1_ref, b_ref, w_ref, o_ref):
  h = jnp.maximum(p0_ref[...] + p1_ref[...] + b_ref[...], 0.0)
  o_ref[...] = jnp.dot(h, w_ref[...], preferred_element_type=jnp.float32)


def _combine(p0, p1, b1, W2p):
  return pl.pallas_call(
      _combine_body,
      grid=(N_NODES // _RB,),
      in_specs=[
          pl.BlockSpec((_RB, NHID), lambda i: (i, 0)),
          pl.BlockSpec((_RB, NHID), lambda i: (i, 0)),
          pl.BlockSpec((1, NHID), lambda i: (0, 0)),
          pl.BlockSpec((NHID, NCLS_PAD), lambda i: (0, 0)),
      ],
      out_specs=pl.BlockSpec((_RB, NCLS_PAD), lambda i: (i, 0)),
      out_shape=jax.ShapeDtypeStruct((N_NODES, NCLS_PAD), jnp.float32),
  )(p0, p1, b1.reshape(1, NHID), W2p)


def _final_body(p0_ref, p1_ref, b_ref, o_ref):
  z = (p0_ref[...] + p1_ref[...])[:, :NCLASS] + b_ref[...]
  z = z - jnp.max(z, axis=1, keepdims=True)
  e = jnp.exp(z)
  o_ref[...] = e / jnp.sum(e, axis=1, keepdims=True)


def _final(p0, p1, b2):
  return pl.pallas_call(
      _final_body,
      grid=(N_NODES // _RB,),
      in_specs=[
          pl.BlockSpec((_RB, NCLS_PAD), lambda i: (i, 0)),
          pl.BlockSpec((_RB, NCLS_PAD), lambda i: (i, 0)),
          pl.BlockSpec((1, NCLASS), lambda i: (0, 0)),
      ],
      out_specs=pl.BlockSpec((_RB, NCLASS), lambda i: (i, 0)),
      out_shape=jax.ShapeDtypeStruct((N_NODES, NCLASS), jnp.float32),
  )(p0, p1, b2.reshape(1, NCLASS))


@jax.jit
def kernel(x, edge_index, edge_weight, W1, b1, W2, b2):
  ei = jnp.pad(edge_index.astype(jnp.int32), ((0, 0), (0, E_PAD - N_EDGES)))
  sd = ei.reshape(2, NW, NCHUNK, CHUNK).transpose(1, 2, 0, 3)
  ewp = jnp.pad(edge_weight, (0, E_PAD - N_EDGES))
  ew = jnp.broadcast_to(ewp.reshape(NW, EDGES_PER_W, 1),
                        (NW, EDGES_PER_W, 16))
  W2p = jnp.pad(W2, ((0, 0), (0, NCLS_PAD - NCLASS)))

  sup1 = _mm1(x, W1)
  p1 = _spmm128(sup1, sd, ew)
  sup2 = _combine(p1[0], p1[1], b1, W2p)
  p2 = _spmm64(sup2, sd, ew)
  return _final(p2[0], p2[1], b2)
